# baseline (speedup 1.0000x reference)
import functools
import jax
import jax.numpy as jnp
from jax import lax
from jax.experimental import pallas as pl
from jax.experimental.pallas import tpu as pltpu
from jax.experimental.pallas import tpu_sc as plsc

BATCH = 1024
K = 100000
LANES = 16
SC_CHK = 192
SC_NCHUNK = 24
SC_QUARTERS = 4
KSC = SC_QUARTERS * SC_NCHUNK * SC_CHK
TCB_K = 2048
TC_FIRST = KSC // TCB_K
K_FULL = KSC + ((K - KSC) // TCB_K) * TCB_K
TC_NBLK = (K_FULL - KSC) // TCB_K
TAIL = K - K_FULL


def _sc_body(stateT_hbm, values_hbm, out_hbm, buf0, buf1, vb0, vb1,
             out_v, sem0, sem1, vsem0, vsem1):
    wid = lax.axis_index("s") * 2 + lax.axis_index("c")
    q = wid // 8
    l = wid % 8
    kbase = q * SC_NCHUNK * SC_CHK
    lane0 = l * 128

    def start_state(c, buf, sem):
        k0 = pl.multiple_of(kbase + c * SC_CHK, 8)
        l0 = pl.multiple_of(lane0, 128)
        pltpu.async_copy(
            stateT_hbm.at[pl.ds(k0, SC_CHK), pl.ds(l0, 128)], buf, sem)

    def start_vals(c, vb, vsem):
        k0 = pl.multiple_of(kbase + c * SC_CHK, 8)
        pltpu.async_copy(values_hbm.at[pl.ds(k0, SC_CHK)], vb, vsem)

    def wait_state(buf, sem):
        pltpu.make_async_copy(
            stateT_hbm.at[pl.ds(0, SC_CHK), pl.ds(0, 128)], buf, sem).wait()

    def wait_vals(vb, vsem):
        pltpu.make_async_copy(values_hbm.at[pl.ds(0, SC_CHK)], vb, vsem).wait()

    def fma_chunk(buf, vb, accs):
        def inner(g, accs):
            vvec = vb[pl.ds(g * LANES, LANES)]
            for p in range(LANES):
                s = vvec[p]
                row = g * LANES + p
                accs = tuple(
                    accs[i] + buf[row, pl.ds(i * LANES, LANES)] * s
                    for i in range(8)
                )
            return accs
        return lax.fori_loop(0, SC_CHK // LANES, inner, accs)

    start_state(0, buf0, sem0)
    start_vals(0, vb0, vsem0)

    def pair_body(i, accs):
        c = i * 2
        start_state(c + 1, buf1, sem1)
        start_vals(c + 1, vb1, vsem1)
        wait_state(buf0, sem0)
        wait_vals(vb0, vsem0)
        accs = fma_chunk(buf0, vb0, accs)
        start_state(c + 2, buf0, sem0)
        start_vals(c + 2, vb0, vsem0)
        wait_state(buf1, sem1)
        wait_vals(vb1, vsem1)
        return fma_chunk(buf1, vb1, accs)

    zero = jnp.zeros((LANES,), jnp.float32)
    accs = lax.fori_loop(0, SC_NCHUNK // 2 - 1, pair_body, (zero,) * 8)

    start_state(SC_NCHUNK - 1, buf1, sem1)
    start_vals(SC_NCHUNK - 1, vb1, vsem1)
    wait_state(buf0, sem0)
    wait_vals(vb0, vsem0)
    accs = fma_chunk(buf0, vb0, accs)
    wait_state(buf1, sem1)
    wait_vals(vb1, vsem1)
    accs = fma_chunk(buf1, vb1, accs)

    for i in range(8):
        out_v[pl.ds(i * LANES, LANES)] = accs[i]
    pltpu.sync_copy(out_v, out_hbm.at[q, l])


def _tc_main_body(vals_ref, state_ref, acc_ref):
    c = pl.program_id(0)

    @pl.when(c == 0)
    def _():
        acc_ref[...] = jnp.zeros_like(acc_ref)

    acc_ref[...] += jnp.dot(vals_ref[...], state_ref[...],
                            preferred_element_type=jnp.float32)


def _tc_combine_body(acc_ref, vtail_ref, stail_ref, sc_ref, out_ref):
    tail = jnp.dot(vtail_ref[...], stail_ref[...],
                   preferred_element_type=jnp.float32)
    scp = jnp.sum(sc_ref[...], axis=0).reshape(1, BATCH)
    out_ref[...] = acc_ref[...] + tail + scp


@jax.jit
def _matvec(state, values):
    stateT = state.T
    vals2d = values.reshape(1, K)
    vals1d = values.reshape(K)

    mesh = plsc.VectorSubcoreMesh(
        core_axis_name="c", subcore_axis_name="s",
        num_cores=2, num_subcores=16,
    )
    sc_fn = pl.kernel(
        _sc_body,
        out_type=jax.ShapeDtypeStruct((SC_QUARTERS, 8, 128), jnp.float32),
        mesh=mesh,
        scratch_types=[
            pltpu.VMEM((SC_CHK, 128), jnp.float32),
            pltpu.VMEM((SC_CHK, 128), jnp.float32),
            pltpu.VMEM((SC_CHK,), jnp.float32),
            pltpu.VMEM((SC_CHK,), jnp.float32),
            pltpu.VMEM((128,), jnp.float32),
            pltpu.SemaphoreType.DMA,
            pltpu.SemaphoreType.DMA,
            pltpu.SemaphoreType.DMA,
            pltpu.SemaphoreType.DMA,
        ],
    )
    part_sc = sc_fn(stateT, vals1d[:KSC])
    part_sc = part_sc.reshape(SC_QUARTERS, BATCH)

    acc_tc = pl.pallas_call(
        _tc_main_body,
        grid=(TC_NBLK,),
        in_specs=[
            pl.BlockSpec((1, TCB_K), lambda c: (0, TC_FIRST + c)),
            pl.BlockSpec((TCB_K, BATCH), lambda c: (TC_FIRST + c, 0)),
        ],
        out_specs=pl.BlockSpec((1, BATCH), lambda c: (0, 0)),
        out_shape=jax.ShapeDtypeStruct((1, BATCH), jnp.float32),
    )(vals2d, stateT)

    vtail = lax.slice(vals2d, (0, K_FULL), (1, K))
    stail = lax.slice(stateT, (K_FULL, 0), (K, BATCH))
    out = pl.pallas_call(
        _tc_combine_body,
        out_shape=jax.ShapeDtypeStruct((1, BATCH), jnp.float32),
    )(acc_tc, vtail, stail, part_sc)
    return out.reshape(BATCH, 1)


def kernel(state, values):
    return _matvec(state, values)


# speedup vs baseline: 1.0053x; 1.0053x over previous
import functools
import jax
import jax.numpy as jnp
from jax import lax
from jax.experimental import pallas as pl
from jax.experimental.pallas import tpu as pltpu
from jax.experimental.pallas import tpu_sc as plsc

BATCH = 1024
K = 100000
LANES = 16
SC_CHK = 192
SC_NCHUNK = 8
SC_QUARTERS = 4
KSC = SC_QUARTERS * SC_NCHUNK * SC_CHK
TCB_K = 2048
TC_FIRST = KSC // TCB_K
K_FULL = KSC + ((K - KSC) // TCB_K) * TCB_K
TC_NBLK = (K_FULL - KSC) // TCB_K
TAIL = K - K_FULL


def _sc_body(stateT_hbm, values_hbm, out_hbm, buf0, buf1, vb0, vb1,
             out_v, sem0, sem1, vsem0, vsem1):
    wid = lax.axis_index("s") * 2 + lax.axis_index("c")
    q = wid // 8
    l = wid % 8
    kbase = q * SC_NCHUNK * SC_CHK
    lane0 = l * 128

    def start_state(c, buf, sem):
        k0 = pl.multiple_of(kbase + c * SC_CHK, 8)
        l0 = pl.multiple_of(lane0, 128)
        pltpu.async_copy(
            stateT_hbm.at[pl.ds(k0, SC_CHK), pl.ds(l0, 128)], buf, sem)

    def start_vals(c, vb, vsem):
        k0 = pl.multiple_of(kbase + c * SC_CHK, 8)
        pltpu.async_copy(values_hbm.at[pl.ds(k0, SC_CHK)], vb, vsem)

    def wait_state(buf, sem):
        pltpu.make_async_copy(
            stateT_hbm.at[pl.ds(0, SC_CHK), pl.ds(0, 128)], buf, sem).wait()

    def wait_vals(vb, vsem):
        pltpu.make_async_copy(values_hbm.at[pl.ds(0, SC_CHK)], vb, vsem).wait()

    def fma_chunk(buf, vb, accs):
        def inner(g, accs):
            vvec = vb[pl.ds(g * LANES, LANES)]
            for p in range(LANES):
                s = vvec[p]
                row = g * LANES + p
                accs = tuple(
                    accs[i] + buf[row, pl.ds(i * LANES, LANES)] * s
                    for i in range(8)
                )
            return accs
        return lax.fori_loop(0, SC_CHK // LANES, inner, accs)

    start_state(0, buf0, sem0)
    start_vals(0, vb0, vsem0)

    def pair_body(i, accs):
        c = i * 2
        start_state(c + 1, buf1, sem1)
        start_vals(c + 1, vb1, vsem1)
        wait_state(buf0, sem0)
        wait_vals(vb0, vsem0)
        accs = fma_chunk(buf0, vb0, accs)
        start_state(c + 2, buf0, sem0)
        start_vals(c + 2, vb0, vsem0)
        wait_state(buf1, sem1)
        wait_vals(vb1, vsem1)
        return fma_chunk(buf1, vb1, accs)

    zero = jnp.zeros((LANES,), jnp.float32)
    accs = lax.fori_loop(0, SC_NCHUNK // 2 - 1, pair_body, (zero,) * 8)

    start_state(SC_NCHUNK - 1, buf1, sem1)
    start_vals(SC_NCHUNK - 1, vb1, vsem1)
    wait_state(buf0, sem0)
    wait_vals(vb0, vsem0)
    accs = fma_chunk(buf0, vb0, accs)
    wait_state(buf1, sem1)
    wait_vals(vb1, vsem1)
    accs = fma_chunk(buf1, vb1, accs)

    for i in range(8):
        out_v[pl.ds(i * LANES, LANES)] = accs[i]
    pltpu.sync_copy(out_v, out_hbm.at[q, l])


def _tc_main_body(vals_ref, state_ref, acc_ref):
    c = pl.program_id(0)

    @pl.when(c == 0)
    def _():
        acc_ref[...] = jnp.zeros_like(acc_ref)

    acc_ref[...] += jnp.dot(vals_ref[...], state_ref[...],
                            preferred_element_type=jnp.float32)


def _tc_combine_body(acc_ref, vtail_ref, stail_ref, sc_ref, out_ref):
    tail = jnp.dot(vtail_ref[...], stail_ref[...],
                   preferred_element_type=jnp.float32)
    scp = jnp.sum(sc_ref[...], axis=0).reshape(1, BATCH)
    out_ref[...] = acc_ref[...] + tail + scp


@jax.jit
def _matvec(state, values):
    stateT = state.T
    vals2d = values.reshape(1, K)
    vals1d = values.reshape(K)

    mesh = plsc.VectorSubcoreMesh(
        core_axis_name="c", subcore_axis_name="s",
        num_cores=2, num_subcores=16,
    )
    sc_fn = pl.kernel(
        _sc_body,
        out_type=jax.ShapeDtypeStruct((SC_QUARTERS, 8, 128), jnp.float32),
        mesh=mesh,
        scratch_types=[
            pltpu.VMEM((SC_CHK, 128), jnp.float32),
            pltpu.VMEM((SC_CHK, 128), jnp.float32),
            pltpu.VMEM((SC_CHK,), jnp.float32),
            pltpu.VMEM((SC_CHK,), jnp.float32),
            pltpu.VMEM((128,), jnp.float32),
            pltpu.SemaphoreType.DMA,
            pltpu.SemaphoreType.DMA,
            pltpu.SemaphoreType.DMA,
            pltpu.SemaphoreType.DMA,
        ],
    )
    part_sc = sc_fn(stateT, vals1d[:KSC])
    part_sc = part_sc.reshape(SC_QUARTERS, BATCH)

    acc_tc = pl.pallas_call(
        _tc_main_body,
        grid=(TC_NBLK,),
        in_specs=[
            pl.BlockSpec((1, TCB_K), lambda c: (0, TC_FIRST + c)),
            pl.BlockSpec((TCB_K, BATCH), lambda c: (TC_FIRST + c, 0)),
        ],
        out_specs=pl.BlockSpec((1, BATCH), lambda c: (0, 0)),
        out_shape=jax.ShapeDtypeStruct((1, BATCH), jnp.float32),
    )(vals2d, stateT)

    vtail = lax.slice(vals2d, (0, K_FULL), (1, K))
    stail = lax.slice(stateT, (K_FULL, 0), (K, BATCH))
    out = pl.pallas_call(
        _tc_combine_body,
        out_shape=jax.ShapeDtypeStruct((1, BATCH), jnp.float32),
    )(acc_tc, vtail, stail, part_sc)
    return out.reshape(BATCH, 1)


def kernel(state, values):
    return _matvec(state, values)
